# baseline (speedup 1.0000x reference)
import jax
import jax.numpy as jnp
from jax.experimental import pallas as pl
from jax.experimental.pallas import tpu as pltpu

B, N, D = 1024, 77, 768
BBLK = 8


def _body(pt_ref, tok_ref, emb_ref, pe_ref, out_ref):
    mask = tok_ref[...] == pt_ref[0]
    out_ref[...] = jnp.where(mask, pe_ref[...][None, :, :], emb_ref[...])


def kernel(tokenized_text, embedded_text, placeholder_token, placeholder_embedding):
    pt = jnp.asarray(placeholder_token, jnp.int32).reshape(1)
    pe = placeholder_embedding.reshape(1, D)
    tok3 = tokenized_text.reshape(B, N, 1)
    grid = (B // BBLK,)
    out = pl.pallas_call(
        _body,
        grid_spec=pltpu.PrefetchScalarGridSpec(
            num_scalar_prefetch=1,
            grid=grid,
            in_specs=[
                pl.BlockSpec((BBLK, N, 1), lambda i, pt_ref: (i, 0, 0)),
                pl.BlockSpec((BBLK, N, D), lambda i, pt_ref: (i, 0, 0)),
                pl.BlockSpec((1, D), lambda i, pt_ref: (0, 0)),
            ],
            out_specs=pl.BlockSpec((BBLK, N, D), lambda i, pt_ref: (i, 0, 0)),
        ),
        out_shape=jax.ShapeDtypeStruct((B, N, D), jnp.float32),
    )(pt, tok3, embedded_text, pe)
    return out
